# baseline (speedup 1.0000x reference)
import functools

import jax
import jax.numpy as jnp
from jax import lax
from jax.experimental import pallas as pl
from jax.experimental.pallas import tpu as pltpu
from jax.experimental.pallas import tpu_sc as plsc

N = 10000
E = 320000
D = 128
HD = 64
G = 64
OUT = 64

CHUNK = 128
HC = CHUNK // 2
CPT = 80
EP = 32 * CPT * CHUNK
NPAD = 10112

BR = 1000
NB = N // BR

_mesh = plsc.VectorSubcoreMesh(core_axis_name="c", subcore_axis_name="s",
                               num_cores=2, num_subcores=16)


@functools.partial(
    pl.kernel,
    out_type=jax.ShapeDtypeStruct((2, NPAD, D), jnp.float32),
    mesh=_mesh,
    scratch_types=[
        pltpu.VMEM((CPT * HC,), jnp.int32),
        pltpu.VMEM((CPT * HC,), jnp.int32),
        [pltpu.VMEM((CHUNK,), jnp.int32) for _ in range(2)],
        pltpu.VMEM((CHUNK,), jnp.int32),
        [pltpu.VMEM((CHUNK, D), jnp.float32) for _ in range(2)],
        pltpu.VMEM_SHARED((NPAD, D), jnp.float32),
        [pltpu.SemaphoreType.DMA for _ in range(2)],
    ],
    compiler_params=pltpu.CompilerParams(use_tc_tiling_on_sc=False),
)
def _sc_aggregate(h_hbm, srcp_hbm, dstp_hbm, zero_hbm, out_hbm,
                  srcp_v, dstp_v, sexp, dexp, bufs, acc_sh, sems):
    cid = lax.axis_index("c")
    sid = lax.axis_index("s")
    wid = sid * 2 + cid
    rpt = NPAD // 16
    pltpu.sync_copy(zero_hbm.at[pl.ds(sid * rpt, rpt)],
                    acc_sh.at[pl.ds(sid * rpt, rpt)])
    pltpu.sync_copy(srcp_hbm.at[wid], srcp_v)
    pltpu.sync_copy(dstp_hbm.at[wid], dstp_v)
    plsc.subcore_barrier()

    def unpack(pk, c, out_ref):
        for j in range(HC // 16):
            w = pk[pl.ds(c * HC + j * 16, 16)]
            out_ref[pl.ds(j * 16, 16)] = w & 0xFFFF
            out_ref[pl.ds(j * 16 + HC, 16)] = lax.shift_right_logical(w, 16)

    def fire(c, b):
        unpack(srcp_v, c, sexp[b])
        pltpu.async_copy(h_hbm.at[sexp[b]], bufs[b], sems[b])

    fire(0, 0)
    fire(1, 1)

    def pair(p, carry):
        c0 = 2 * p
        for b in range(2):
            c = c0 + b
            pltpu.make_async_copy(h_hbm.at[sexp[b]], bufs[b], sems[b]).wait()
            unpack(dstp_v, c, dexp)
            pltpu.sync_copy(bufs[b], acc_sh.at[dexp], add=True)

            @pl.when(c + 2 < CPT)
            def _(c=c, b=b):
                fire(c + 2, b)
        return carry

    lax.fori_loop(0, CPT // 2, pair, 0)
    plsc.subcore_barrier()
    pltpu.sync_copy(acc_sh.at[pl.ds(sid * rpt, rpt)],
                    out_hbm.at[cid, pl.ds(sid * rpt, rpt)])


def _mlp_body(x_ref, a_ref, w1_ref, b1_ref, w2_ref, b2_ref,
              act_ref, sum_ref, sq_ref):
    i = pl.program_id(0)
    h = x_ref[...] + a_ref[0] + a_ref[1]
    z = jnp.dot(h, w1_ref[...], preferred_element_type=jnp.float32)
    z = jnp.maximum(z + b1_ref[...], 0.0)
    z = jnp.dot(z, w2_ref[...], preferred_element_type=jnp.float32)
    z = jnp.maximum(z + b2_ref[...], 0.0)
    act_ref[...] = z

    @pl.when(i == 0)
    def _init():
        sum_ref[...] = jnp.zeros_like(sum_ref)
        sq_ref[...] = jnp.zeros_like(sq_ref)

    sum_ref[...] += jnp.sum(z, axis=0, keepdims=True)
    sq_ref[...] += jnp.sum(z * z, axis=0, keepdims=True)


def _mlp(x, agg, w1, b1, w2, b2):
    return pl.pallas_call(
        _mlp_body,
        grid=(NB,),
        in_specs=[
            pl.BlockSpec((BR, D), lambda i: (i, 0)),
            pl.BlockSpec((2, BR, D), lambda i: (0, i, 0)),
            pl.BlockSpec((D, D), lambda i: (0, 0)),
            pl.BlockSpec((1, D), lambda i: (0, 0)),
            pl.BlockSpec((D, D), lambda i: (0, 0)),
            pl.BlockSpec((1, D), lambda i: (0, 0)),
        ],
        out_specs=[
            pl.BlockSpec((BR, D), lambda i: (i, 0)),
            pl.BlockSpec((1, D), lambda i: (0, 0)),
            pl.BlockSpec((1, D), lambda i: (0, 0)),
        ],
        out_shape=[
            jax.ShapeDtypeStruct((N, D), jnp.float32),
            jax.ShapeDtypeStruct((1, D), jnp.float32),
            jax.ShapeDtypeStruct((1, D), jnp.float32),
        ],
    )(x, agg, w1, b1, w2, b2)


def _bn_body(act_ref, sum_ref, sq_ref, g_ref, be_ref, out_ref):
    mu = sum_ref[...] * (1.0 / N)
    var = sq_ref[...] * (1.0 / N) - mu * mu
    a = g_ref[...] * lax.rsqrt(var + 1e-5)
    out_ref[...] = act_ref[...] * a + (be_ref[...] - mu * a)


def _bn_apply(act, s, q, g, be):
    return pl.pallas_call(
        _bn_body,
        grid=(NB,),
        in_specs=[
            pl.BlockSpec((BR, D), lambda i: (i, 0)),
            pl.BlockSpec((1, D), lambda i: (0, 0)),
            pl.BlockSpec((1, D), lambda i: (0, 0)),
            pl.BlockSpec((1, D), lambda i: (0, 0)),
            pl.BlockSpec((1, D), lambda i: (0, 0)),
        ],
        out_specs=pl.BlockSpec((BR, D), lambda i: (i, 0)),
        out_shape=jax.ShapeDtypeStruct((N, D), jnp.float32),
    )(act, s, q, g, be)


def _head_body(act_ref, batch_ref, sum_ref, sq_ref, g_ref, be_ref,
               w1_ref, b1_ref, mg_ref, mbe_ref, w2_ref, b2_ref,
               logp_ref, soft_ref, last_ref, pooled_ref, cnt_ref):
    i = pl.program_id(0)

    @pl.when(i == 0)
    def _init():
        pooled_ref[...] = jnp.zeros_like(pooled_ref)
        cnt_ref[...] = jnp.zeros_like(cnt_ref)

    b = batch_ref[0]
    gids = lax.broadcasted_iota(jnp.int32, (G, BR), 0)
    oh = (gids == b).astype(jnp.float32)
    pooled_ref[...] += jnp.dot(oh, act_ref[...],
                               preferred_element_type=jnp.float32)
    cnt_ref[...] += jnp.sum(oh, axis=1, keepdims=True)

    @pl.when(i == NB - 1)
    def _fin():
        mu = sum_ref[...] * (1.0 / N)
        var = sq_ref[...] * (1.0 / N) - mu * mu
        a = g_ref[...] * lax.rsqrt(var + 1e-5)
        c = be_ref[...] - mu * a
        cnt = cnt_ref[:, :1]
        pooled = (pooled_ref[...] * a + cnt * c) / jnp.maximum(cnt, 1.0)
        z = jnp.dot(pooled, w1_ref[...],
                    preferred_element_type=jnp.float32) + b1_ref[...]
        zmu = jnp.mean(z, axis=0, keepdims=True)
        zc = z - zmu
        zvar = jnp.mean(zc * zc, axis=0, keepdims=True)
        z = mg_ref[...] * zc * lax.rsqrt(zvar + 1e-5) + mbe_ref[...]
        z = jnp.maximum(z, 0.0)
        last = jnp.dot(z, w2_ref[...],
                       preferred_element_type=jnp.float32) + b2_ref[...]
        m = jnp.max(last, axis=-1, keepdims=True)
        ex = jnp.exp(last - m)
        se = jnp.sum(ex, axis=-1, keepdims=True)
        logp = last - m - jnp.log(se)
        last_ref[...] = last
        logp_ref[...] = logp
        soft_ref[...] = ex / se


def _head(act, batch3, s, q, g, be, w1, b1, mg, mbe, w2, b2):
    vec = pl.BlockSpec((1, D), lambda i: (0, 0))
    return pl.pallas_call(
        _head_body,
        grid=(NB,),
        in_specs=[
            pl.BlockSpec((BR, D), lambda i: (i, 0)),
            pl.BlockSpec((1, 1, BR), lambda i: (i, 0, 0)),
            vec, vec, vec, vec,
            pl.BlockSpec((D, D), lambda i: (0, 0)),
            vec, vec, vec,
            pl.BlockSpec((D, OUT), lambda i: (0, 0)),
            pl.BlockSpec((1, OUT), lambda i: (0, 0)),
        ],
        out_specs=[
            pl.BlockSpec((G, OUT), lambda i: (0, 0)),
            pl.BlockSpec((G, OUT), lambda i: (0, 0)),
            pl.BlockSpec((G, OUT), lambda i: (0, 0)),
        ],
        out_shape=[
            jax.ShapeDtypeStruct((G, OUT), jnp.float32),
            jax.ShapeDtypeStruct((G, OUT), jnp.float32),
            jax.ShapeDtypeStruct((G, OUT), jnp.float32),
        ],
        scratch_shapes=[
            pltpu.VMEM((G, D), jnp.float32),
            pltpu.VMEM((G, D), jnp.float32),
        ],
    )(act, batch3, s, q, g, be, w1, b1, mg, mbe, w2, b2)


def kernel(x, edge_index, edge_weight, batch,
           l0_W1, l0_b1, l0_W2, l0_b2, l0_g, l0_be,
           l1_W1, l1_b1, l1_W2, l1_b2, l1_g, l1_be,
           m_W1, m_b1, m_g, m_be, m_W2, m_b2):
    del edge_weight
    pad = EP - E

    def pack(a, fill):
        a = jnp.concatenate([a, jnp.full((pad,), fill, jnp.int32)])
        a = a.reshape(32, CPT, 2, HC)
        return (a[:, :, 0, :] | (a[:, :, 1, :] << 16)).reshape(32, CPT * HC)

    src = pack(edge_index[0], 0)
    dst = pack(edge_index[1], N)
    zero = jnp.zeros((NPAD, D), jnp.float32)

    r1 = lambda v: v.reshape(1, -1)
    agg0 = _sc_aggregate(x, src, dst, zero)
    act0, s0, q0 = _mlp(x, agg0, l0_W1, r1(l0_b1), l0_W2, r1(l0_b2))
    h0 = _bn_apply(act0, s0, q0, r1(l0_g), r1(l0_be))
    agg1 = _sc_aggregate(h0, src, dst, zero)
    act1, s1, q1 = _mlp(h0, agg1, l1_W1, r1(l1_b1), l1_W2, r1(l1_b2))
    batch3 = batch.reshape(NB, 1, BR)
    logp, soft, last = _head(act1, batch3, s1, q1, r1(l1_g), r1(l1_be),
                             m_W1, r1(m_b1), r1(m_g), r1(m_be),
                             m_W2, r1(m_b2))
    return (logp, soft, last)


# speedup vs baseline: 1.5289x; 1.5289x over previous
import functools

import jax
import jax.numpy as jnp
from jax import lax
from jax.experimental import pallas as pl
from jax.experimental.pallas import tpu as pltpu
from jax.experimental.pallas import tpu_sc as plsc

N = 10000
E = 320000
D = 128
HD = 64
G = 64
OUT = 64

CHUNK = 128
CPT = 160
NBUF = 5
EP = 16 * CPT * CHUNK
NPAD = 10112

BR = 1000
NB = N // BR

_mesh = plsc.VectorSubcoreMesh(core_axis_name="c", subcore_axis_name="s",
                               num_cores=2, num_subcores=16)


@functools.partial(
    pl.kernel,
    out_type=jax.ShapeDtypeStruct((2, NPAD, HD), jnp.float32),
    mesh=_mesh,
    scratch_types=[
        pltpu.VMEM((CPT * CHUNK,), jnp.int32),
        pltpu.VMEM((CPT, CHUNK), jnp.int32),
        [pltpu.VMEM((CHUNK, HD), jnp.float32) for _ in range(NBUF)],
        pltpu.VMEM_SHARED((NPAD, HD), jnp.float32),
        [pltpu.SemaphoreType.DMA for _ in range(NBUF)],
    ],
    compiler_params=pltpu.CompilerParams(use_tc_tiling_on_sc=False),
)
def _sc_aggregate(h2_hbm, src_hbm, dst_hbm, zero_hbm, out_hbm,
                  src_v, dst_v, bufs, acc_sh, sems):
    cid = lax.axis_index("c")
    sid = lax.axis_index("s")
    rpt = NPAD // 16
    half = h2_hbm.at[cid]
    pltpu.sync_copy(zero_hbm.at[pl.ds(sid * rpt, rpt)],
                    acc_sh.at[pl.ds(sid * rpt, rpt)])
    pltpu.sync_copy(src_hbm.at[sid], src_v)
    pltpu.sync_copy(dst_hbm.at[sid], dst_v)
    plsc.subcore_barrier()

    def fire(c, b):
        pltpu.async_copy(half.at[src_v.at[pl.ds(c * CHUNK, CHUNK)]],
                         bufs[b], sems[b])

    for b in range(NBUF):
        fire(b, b)

    def group(g, carry):
        c0 = NBUF * g
        for b in range(NBUF):
            c = c0 + b
            pltpu.make_async_copy(half.at[src_v.at[pl.ds(c * CHUNK, CHUNK)]],
                                  bufs[b], sems[b]).wait()
            pltpu.sync_copy(bufs[b], acc_sh.at[dst_v.at[c]], add=True)

            @pl.when(c + NBUF < CPT)
            def _(c=c, b=b):
                fire(c + NBUF, b)
        return carry

    lax.fori_loop(0, CPT // NBUF, group, 0)
    plsc.subcore_barrier()
    pltpu.sync_copy(acc_sh.at[pl.ds(sid * rpt, rpt)],
                    out_hbm.at[cid, pl.ds(sid * rpt, rpt)])


def _mlp_body(x2_ref, a_ref, w1_ref, b1_ref, w2_ref, b2_ref,
              act_ref, sum_ref, sq_ref):
    i = pl.program_id(0)
    h = (jnp.concatenate([x2_ref[0], x2_ref[1]], axis=-1)
         + jnp.concatenate([a_ref[0], a_ref[1]], axis=-1))
    z = jnp.dot(h, w1_ref[...], preferred_element_type=jnp.float32)
    z = jnp.maximum(z + b1_ref[...], 0.0)
    z = jnp.dot(z, w2_ref[...], preferred_element_type=jnp.float32)
    z = jnp.maximum(z + b2_ref[...], 0.0)
    act_ref[...] = z

    @pl.when(i == 0)
    def _init():
        sum_ref[...] = jnp.zeros_like(sum_ref)
        sq_ref[...] = jnp.zeros_like(sq_ref)

    sum_ref[...] += jnp.sum(z, axis=0, keepdims=True)
    sq_ref[...] += jnp.sum(z * z, axis=0, keepdims=True)


def _mlp(x2, agg, w1, b1, w2, b2):
    return pl.pallas_call(
        _mlp_body,
        grid=(NB,),
        in_specs=[
            pl.BlockSpec((2, BR, HD), lambda i: (0, i, 0)),
            pl.BlockSpec((2, BR, HD), lambda i: (0, i, 0)),
            pl.BlockSpec((D, D), lambda i: (0, 0)),
            pl.BlockSpec((1, D), lambda i: (0, 0)),
            pl.BlockSpec((D, D), lambda i: (0, 0)),
            pl.BlockSpec((1, D), lambda i: (0, 0)),
        ],
        out_specs=[
            pl.BlockSpec((BR, D), lambda i: (i, 0)),
            pl.BlockSpec((1, D), lambda i: (0, 0)),
            pl.BlockSpec((1, D), lambda i: (0, 0)),
        ],
        out_shape=[
            jax.ShapeDtypeStruct((N, D), jnp.float32),
            jax.ShapeDtypeStruct((1, D), jnp.float32),
            jax.ShapeDtypeStruct((1, D), jnp.float32),
        ],
    )(x2, agg, w1, b1, w2, b2)


def _mlp1_body(x2_ref, a_ref, batch_ref, w1_ref, b1_ref, w2_ref, b2_ref,
               sum_ref, sq_ref, pooled_ref, cnt_ref):
    i = pl.program_id(0)
    h = (jnp.concatenate([x2_ref[0], x2_ref[1]], axis=-1)
         + jnp.concatenate([a_ref[0], a_ref[1]], axis=-1))
    z = jnp.dot(h, w1_ref[...], preferred_element_type=jnp.float32)
    z = jnp.maximum(z + b1_ref[...], 0.0)
    z = jnp.dot(z, w2_ref[...], preferred_element_type=jnp.float32)
    z = jnp.maximum(z + b2_ref[...], 0.0)

    @pl.when(i == 0)
    def _init():
        sum_ref[...] = jnp.zeros_like(sum_ref)
        sq_ref[...] = jnp.zeros_like(sq_ref)
        pooled_ref[...] = jnp.zeros_like(pooled_ref)
        cnt_ref[...] = jnp.zeros_like(cnt_ref)

    sum_ref[...] += jnp.sum(z, axis=0, keepdims=True)
    sq_ref[...] += jnp.sum(z * z, axis=0, keepdims=True)
    b = batch_ref[0]
    gids = lax.broadcasted_iota(jnp.int32, (G, BR), 0)
    oh = (gids == b).astype(jnp.float32)
    pooled_ref[...] += jnp.dot(oh, z, preferred_element_type=jnp.float32)
    cnt_ref[...] += jnp.sum(oh, axis=1, keepdims=True)


def _mlp1(x2, agg, batch3, w1, b1, w2, b2):
    con = pl.BlockSpec((1, D), lambda i: (0, 0))
    return pl.pallas_call(
        _mlp1_body,
        grid=(NB,),
        in_specs=[
            pl.BlockSpec((2, BR, HD), lambda i: (0, i, 0)),
            pl.BlockSpec((2, BR, HD), lambda i: (0, i, 0)),
            pl.BlockSpec((1, 1, BR), lambda i: (i, 0, 0)),
            pl.BlockSpec((D, D), lambda i: (0, 0)),
            con,
            pl.BlockSpec((D, D), lambda i: (0, 0)),
            con,
        ],
        out_specs=[
            con, con,
            pl.BlockSpec((G, D), lambda i: (0, 0)),
            pl.BlockSpec((G, D), lambda i: (0, 0)),
        ],
        out_shape=[
            jax.ShapeDtypeStruct((1, D), jnp.float32),
            jax.ShapeDtypeStruct((1, D), jnp.float32),
            jax.ShapeDtypeStruct((G, D), jnp.float32),
            jax.ShapeDtypeStruct((G, D), jnp.float32),
        ],
    )(x2, agg, batch3, w1, b1, w2, b2)


def _bn_body(act_ref, sum_ref, sq_ref, g_ref, be_ref, out_ref):
    mu = sum_ref[...] * (1.0 / N)
    var = sq_ref[...] * (1.0 / N) - mu * mu
    a = g_ref[...] * lax.rsqrt(var + 1e-5)
    z = act_ref[...] * a + (be_ref[...] - mu * a)
    out_ref[0] = z[:, :HD]
    out_ref[1] = z[:, HD:]


def _bn_apply(act, s, q, g, be):
    return pl.pallas_call(
        _bn_body,
        grid=(NB,),
        in_specs=[
            pl.BlockSpec((BR, D), lambda i: (i, 0)),
            pl.BlockSpec((1, D), lambda i: (0, 0)),
            pl.BlockSpec((1, D), lambda i: (0, 0)),
            pl.BlockSpec((1, D), lambda i: (0, 0)),
            pl.BlockSpec((1, D), lambda i: (0, 0)),
        ],
        out_specs=pl.BlockSpec((2, BR, HD), lambda i: (0, i, 0)),
        out_shape=jax.ShapeDtypeStruct((2, N, HD), jnp.float32),
    )(act, s, q, g, be)


def _head_body(pooled_ref, cnt_ref, sum_ref, sq_ref, g_ref, be_ref,
               w1_ref, b1_ref, mg_ref, mbe_ref, w2_ref, b2_ref,
               logp_ref, soft_ref, last_ref):
    mu = sum_ref[...] * (1.0 / N)
    var = sq_ref[...] * (1.0 / N) - mu * mu
    a = g_ref[...] * lax.rsqrt(var + 1e-5)
    c = be_ref[...] - mu * a
    cnt = cnt_ref[:, :1]
    pooled = (pooled_ref[...] * a + cnt * c) / jnp.maximum(cnt, 1.0)
    z = jnp.dot(pooled, w1_ref[...],
                preferred_element_type=jnp.float32) + b1_ref[...]
    zmu = jnp.mean(z, axis=0, keepdims=True)
    zc = z - zmu
    zvar = jnp.mean(zc * zc, axis=0, keepdims=True)
    z = mg_ref[...] * zc * lax.rsqrt(zvar + 1e-5) + mbe_ref[...]
    z = jnp.maximum(z, 0.0)
    last = jnp.dot(z, w2_ref[...],
                   preferred_element_type=jnp.float32) + b2_ref[...]
    m = jnp.max(last, axis=-1, keepdims=True)
    ex = jnp.exp(last - m)
    se = jnp.sum(ex, axis=-1, keepdims=True)
    logp = last - m - jnp.log(se)
    last_ref[...] = last
    logp_ref[...] = logp
    soft_ref[...] = ex / se


def _head(pooled, cnt, s, q, g, be, w1, b1, mg, mbe, w2, b2):
    return pl.pallas_call(
        _head_body,
        out_shape=[
            jax.ShapeDtypeStruct((G, OUT), jnp.float32),
            jax.ShapeDtypeStruct((G, OUT), jnp.float32),
            jax.ShapeDtypeStruct((G, OUT), jnp.float32),
        ],
    )(pooled, cnt, s, q, g, be, w1, b1, mg, mbe, w2, b2)


def kernel(x, edge_index, edge_weight, batch,
           l0_W1, l0_b1, l0_W2, l0_b2, l0_g, l0_be,
           l1_W1, l1_b1, l1_W2, l1_b2, l1_g, l1_be,
           m_W1, m_b1, m_g, m_be, m_W2, m_b2):
    del edge_weight
    pad = EP - E
    src = jnp.concatenate(
        [edge_index[0], jnp.zeros((pad,), jnp.int32)]).reshape(16, CPT * CHUNK)
    dst = jnp.concatenate(
        [edge_index[1], jnp.full((pad,), N, jnp.int32)]).reshape(16, CPT, CHUNK)
    zero = jnp.zeros((NPAD, HD), jnp.float32)
    x2 = jnp.stack([x[:, :HD], x[:, HD:]])

    r1 = lambda v: v.reshape(1, -1)
    agg0 = _sc_aggregate(x2, src, dst, zero)
    act0, s0, q0 = _mlp(x2, agg0, l0_W1, r1(l0_b1), l0_W2, r1(l0_b2))
    h2 = _bn_apply(act0, s0, q0, r1(l0_g), r1(l0_be))
    agg1 = _sc_aggregate(h2, src, dst, zero)
    batch3 = batch.reshape(NB, 1, BR)
    s1, q1, pooled, cnt = _mlp1(h2, agg1, batch3,
                                l1_W1, r1(l1_b1), l1_W2, r1(l1_b2))
    logp, soft, last = _head(pooled, cnt, s1, q1, r1(l1_g), r1(l1_be),
                             m_W1, r1(m_b1), r1(m_g), r1(m_be),
                             m_W2, r1(m_b2))
    return (logp, soft, last)
